# Initial kernel scaffold; baseline (speedup 1.0000x reference)
import functools

import jax
import jax.numpy as jnp
from jax import lax
from jax.experimental import pallas as pl
from jax.experimental.pallas import tpu as pltpu
from jax.experimental.pallas import tpu_sc as plsc

NC = 2
NS = 16
LANES = 16
CHUNK = 128


def _matmul_body(x_ref, w_ref, o_ref):
    o_ref[...] = jnp.dot(x_ref[...], w_ref[...],
                         preferred_element_type=jnp.float32)


def _combine_body(a_ref, b_ref, bias_ref, o_ref):
    o_ref[...] = a_ref[0] + b_ref[0] + bias_ref[...]


def _scale_rows(rows_ref, w_smem, n_edges, d):
    nv = d // LANES

    @pl.loop(0, n_edges)
    def _(e):
        w = w_smem[e]
        wv = jnp.full((LANES,), w, dtype=jnp.float32)
        for j in range(nv):
            sl = pl.ds(j * LANES, LANES)
            rows_ref[e, sl] = rows_ref[e, sl] * wv


def _make_spmm(n_nodes, n_edges, d):
    per_tile = n_edges // (NC * NS)
    n_full = per_tile // CHUNK
    tail = per_tile - n_full * CHUNK
    rows_per_tile = n_nodes // NS
    zrows = rows_per_tile
    for cand in (128, 125, 100, 64, 50, 25, 8, 5, 1):
        if rows_per_tile % cand == 0:
            zrows = cand
            break
    nz = rows_per_tile // zrows
    nv = d // LANES

    mesh = plsc.VectorSubcoreMesh(core_axis_name="c", subcore_axis_name="s")

    scratch = [
        pltpu.VMEM((CHUNK,), jnp.int32),
        pltpu.VMEM((CHUNK,), jnp.int32),
        pltpu.VMEM((CHUNK, d), jnp.float32),
        pltpu.SMEM((CHUNK,), jnp.float32),
        pltpu.VMEM((zrows, d), jnp.float32),
        pltpu.VMEM_SHARED((n_nodes, d), jnp.float32),
        pltpu.SemaphoreType.DMA,
    ]
    if tail:
        scratch += [
            pltpu.VMEM((tail,), jnp.int32),
            pltpu.VMEM((tail,), jnp.int32),
            pltpu.VMEM((tail, d), jnp.float32),
            pltpu.SMEM((tail,), jnp.float32),
        ]

    def body(h_hbm, col_hbm, row_hbm, w_hbm, out_hbm,
             colv, rowv, rows_v, w_sm, zblk, agg, sem,
             *tail_refs):
        c = lax.axis_index("c")
        s = lax.axis_index("s")

        @pl.loop(0, zrows)
        def _(i):
            for j in range(nv):
                zblk[i, pl.ds(j * LANES, LANES)] = jnp.zeros(
                    (LANES,), jnp.float32)

        base_row = s * rows_per_tile
        for q in range(nz):
            pltpu.sync_copy(zblk, agg.at[pl.ds(base_row + q * zrows, zrows)])
        plsc.subcore_barrier()

        eb = (c * NS + s) * per_tile

        @pl.loop(0, n_full)
        def _(k):
            off = eb + k * CHUNK
            pltpu.sync_copy(col_hbm.at[pl.ds(off, CHUNK)], colv)
            pltpu.sync_copy(w_hbm.at[pl.ds(off, CHUNK)], w_sm)
            pltpu.async_copy(h_hbm.at[colv], rows_v, sem).wait()
            pltpu.sync_copy(row_hbm.at[pl.ds(off, CHUNK)], rowv)
            _scale_rows(rows_v, w_sm, CHUNK, d)
            pltpu.sync_copy(rows_v, agg.at[rowv], add=True)

        if tail:
            tcol, trow, trows, tw = tail_refs
            off = eb + n_full * CHUNK
            pltpu.sync_copy(col_hbm.at[pl.ds(off, tail)], tcol)
            pltpu.sync_copy(w_hbm.at[pl.ds(off, tail)], tw)
            pltpu.async_copy(h_hbm.at[tcol], trows, sem).wait()
            pltpu.sync_copy(row_hbm.at[pl.ds(off, tail)], trow)
            _scale_rows(trows, tw, tail, d)
            pltpu.sync_copy(trows, agg.at[trow], add=True)

        plsc.subcore_barrier()
        pltpu.sync_copy(agg.at[pl.ds(base_row, rows_per_tile)],
                        out_hbm.at[c, pl.ds(base_row, rows_per_tile)])

    return pl.kernel(
        body,
        out_type=jax.ShapeDtypeStruct((NC, n_nodes, d), jnp.float32),
        mesh=mesh,
        scratch_types=scratch,
    )


@jax.jit
def kernel(x, edge_index, edge_weight, weight, bias):
    n, d_in = x.shape
    d = weight.shape[1]
    n_edges = edge_weight.shape[0]

    blk = 1000 if n % 1000 == 0 else n
    h = pl.pallas_call(
        _matmul_body,
        grid=(n // blk,),
        in_specs=[
            pl.BlockSpec((blk, d_in), lambda i: (i, 0)),
            pl.BlockSpec((d_in, d), lambda i: (0, 0)),
        ],
        out_specs=pl.BlockSpec((blk, d), lambda i: (i, 0)),
        out_shape=jax.ShapeDtypeStruct((n, d), jnp.float32),
    )(x, weight)

    ei = edge_index.astype(jnp.int32)
    row = ei[0]
    col = ei[1]
    ew = edge_weight.astype(jnp.float32)

    partials = _make_spmm(n, n_edges, d)(h, col, row, ew)

    out = pl.pallas_call(
        _combine_body,
        grid=(n // blk,),
        in_specs=[
            pl.BlockSpec((1, blk, d), lambda i: (0, i, 0)),
            pl.BlockSpec((1, blk, d), lambda i: (1, i, 0)),
            pl.BlockSpec((d,), lambda i: (0,)),
        ],
        out_specs=pl.BlockSpec((blk, d), lambda i: (i, 0)),
        out_shape=jax.ShapeDtypeStruct((n, d), jnp.float32),
    )(partials, partials, bias)
    return out


# speedup vs baseline: 5.4057x; 5.4057x over previous
import functools

import jax
import jax.numpy as jnp
from jax import lax
from jax.experimental import pallas as pl
from jax.experimental.pallas import tpu as pltpu
from jax.experimental.pallas import tpu_sc as plsc

NC = 2
NS = 16
LANES = 16
CHUNK = 128


def _matmul_body(x_ref, w_ref, o_ref):
    o_ref[...] = jnp.dot(x_ref[...], w_ref[...],
                         preferred_element_type=jnp.float32)


def _combine_body(a_ref, b_ref, bias_ref, o_ref):
    o_ref[...] = a_ref[0] + b_ref[0] + bias_ref[...]


def _scale_rows(rows_ref, w_ref, n_edges, d):
    nv = d // LANES
    assert n_edges % LANES == 0

    @pl.loop(0, n_edges // LANES)
    def _(g):
        wv16 = w_ref[pl.ds(g * LANES, LANES)]
        for l in range(LANES):
            wb = jnp.full((LANES,), wv16[l], dtype=jnp.float32)
            e = g * LANES + l
            for j in range(nv):
                sl = pl.ds(j * LANES, LANES)
                rows_ref[e, sl] = rows_ref[e, sl] * wb


def _make_spmm(n_nodes, n_edges, d):
    per_tile = n_edges // (NC * NS)
    n_full = per_tile // CHUNK
    tail = per_tile - n_full * CHUNK
    rows_per_tile = (n_nodes // NS) // 8 * 8
    leftover = n_nodes - rows_per_tile * NS
    assert leftover % 8 == 0
    zrows = rows_per_tile
    for cand in (256, 248, 240, 232, 224, 216, 208, 200, 192, 184, 176,
                 168, 160, 152, 144, 136, 128, 120, 112, 104, 96, 88, 80,
                 72, 64, 56, 48, 40, 32, 24, 16, 8):
        if rows_per_tile % cand == 0:
            zrows = cand
            break
    nz = rows_per_tile // zrows
    assert leftover <= zrows
    nv = d // LANES

    mesh = plsc.VectorSubcoreMesh(core_axis_name="c", subcore_axis_name="s")

    scratch = [
        pltpu.VMEM((CHUNK,), jnp.int32),
        pltpu.VMEM((CHUNK,), jnp.int32),
        pltpu.VMEM((CHUNK, d), jnp.float32),
        pltpu.VMEM((CHUNK,), jnp.float32),
        pltpu.VMEM((zrows, d), jnp.float32),
        pltpu.VMEM_SHARED((n_nodes, d), jnp.float32),
        pltpu.SemaphoreType.DMA,
    ]
    if tail:
        scratch += [
            pltpu.VMEM((tail,), jnp.int32),
            pltpu.VMEM((tail,), jnp.int32),
            pltpu.VMEM((tail, d), jnp.float32),
            pltpu.VMEM((tail,), jnp.float32),
        ]

    def body(h_hbm, col_hbm, row_hbm, w_hbm, out_hbm,
             colv, rowv, rows_v, w_sm, zblk, agg, sem,
             *tail_refs):
        c = lax.axis_index("c")
        s = lax.axis_index("s")

        @pl.loop(0, zrows)
        def _(i):
            for j in range(nv):
                zblk[i, pl.ds(j * LANES, LANES)] = jnp.zeros(
                    (LANES,), jnp.float32)

        base_row = s * rows_per_tile
        for q in range(nz):
            pltpu.sync_copy(zblk, agg.at[pl.ds(base_row + q * zrows, zrows)])
        if leftover:
            @pl.when(s == NS - 1)
            def _():
                pltpu.sync_copy(
                    zblk.at[pl.ds(0, leftover)],
                    agg.at[pl.ds(rows_per_tile * NS, leftover)])
        plsc.subcore_barrier()

        eb = (c * NS + s) * per_tile

        @pl.loop(0, n_full)
        def _(k):
            off = eb + k * CHUNK
            pltpu.sync_copy(col_hbm.at[pl.ds(off, CHUNK)], colv)
            pltpu.sync_copy(w_hbm.at[pl.ds(off, CHUNK)], w_sm)
            pltpu.async_copy(h_hbm.at[colv], rows_v, sem).wait()
            pltpu.sync_copy(row_hbm.at[pl.ds(off, CHUNK)], rowv)
            _scale_rows(rows_v, w_sm, CHUNK, d)
            pltpu.sync_copy(rows_v, agg.at[rowv], add=True)

        if tail:
            tcol, trow, trows, tw = tail_refs
            off = eb + n_full * CHUNK
            pltpu.sync_copy(col_hbm.at[pl.ds(off, tail)], tcol)
            pltpu.sync_copy(w_hbm.at[pl.ds(off, tail)], tw)
            pltpu.async_copy(h_hbm.at[tcol], trows, sem).wait()
            pltpu.sync_copy(row_hbm.at[pl.ds(off, tail)], trow)
            _scale_rows(trows, tw, tail, d)
            pltpu.sync_copy(trows, agg.at[trow], add=True)

        plsc.subcore_barrier()
        pltpu.sync_copy(agg.at[pl.ds(base_row, rows_per_tile)],
                        out_hbm.at[c, pl.ds(base_row, rows_per_tile)])
        if leftover:
            @pl.when(s == NS - 1)
            def _():
                pltpu.sync_copy(
                    agg.at[pl.ds(rows_per_tile * NS, leftover)],
                    out_hbm.at[c, pl.ds(rows_per_tile * NS, leftover)])

    return pl.kernel(
        body,
        out_type=jax.ShapeDtypeStruct((NC, n_nodes, d), jnp.float32),
        mesh=mesh,
        scratch_types=scratch,
    )


@jax.jit
def kernel(x, edge_index, edge_weight, weight, bias):
    n, d_in = x.shape
    d = weight.shape[1]
    n_edges = edge_weight.shape[0]

    blk = 1000 if n % 1000 == 0 else n
    h = pl.pallas_call(
        _matmul_body,
        grid=(n // blk,),
        in_specs=[
            pl.BlockSpec((blk, d_in), lambda i: (i, 0)),
            pl.BlockSpec((d_in, d), lambda i: (0, 0)),
        ],
        out_specs=pl.BlockSpec((blk, d), lambda i: (i, 0)),
        out_shape=jax.ShapeDtypeStruct((n, d), jnp.float32),
    )(x, weight)

    ei = edge_index.astype(jnp.int32)
    row = ei[0]
    col = ei[1]
    ew = edge_weight.astype(jnp.float32)

    partials = _make_spmm(n, n_edges, d)(h, col, row, ew)

    out = pl.pallas_call(
        _combine_body,
        grid=(n // blk,),
        in_specs=[
            pl.BlockSpec((1, blk, d), lambda i: (0, i, 0)),
            pl.BlockSpec((1, blk, d), lambda i: (1, i, 0)),
            pl.BlockSpec((d,), lambda i: (0,)),
        ],
        out_specs=pl.BlockSpec((blk, d), lambda i: (i, 0)),
        out_shape=jax.ShapeDtypeStruct((n, d), jnp.float32),
    )(partials, partials, bias)
    return out


# speedup vs baseline: 10.2041x; 1.8877x over previous
import functools

import jax
import jax.numpy as jnp
from jax import lax
from jax.experimental import pallas as pl
from jax.experimental.pallas import tpu as pltpu
from jax.experimental.pallas import tpu_sc as plsc

NC = 2
NS = 16
LANES = 16
CHUNK = 128


def _matmul_body(x_ref, w_ref, o_ref):
    o_ref[...] = jnp.dot(x_ref[...], w_ref[...],
                         preferred_element_type=jnp.float32)


def _combine_body(a_ref, b_ref, bias_ref, o_ref):
    o_ref[...] = a_ref[0] + b_ref[0] + bias_ref[...]


def _scale_rows(rows_ref, w_ref, n_edges, d):
    nv = d // LANES
    assert n_edges % LANES == 0

    @pl.loop(0, n_edges // LANES)
    def _(g):
        wv16 = w_ref[pl.ds(g * LANES, LANES)]
        for l in range(LANES):
            wb = jnp.full((LANES,), wv16[l], dtype=jnp.float32)
            e = g * LANES + l
            for j in range(nv):
                sl = pl.ds(j * LANES, LANES)
                rows_ref[e, sl] = rows_ref[e, sl] * wb


def _make_spmm(n_nodes, n_edges, d):
    per_tile = n_edges // (NC * NS)
    n_full = per_tile // CHUNK
    tail = per_tile - n_full * CHUNK
    rows_per_tile = (n_nodes // NS) // 8 * 8
    leftover = n_nodes - rows_per_tile * NS
    assert leftover % 8 == 0
    zrows = rows_per_tile
    for cand in (64, 56, 48, 40, 32, 24, 16, 8):
        if rows_per_tile % cand == 0:
            zrows = cand
            break
    nz = rows_per_tile // zrows
    assert leftover <= zrows
    nv = d // LANES

    NBUF = 2
    pipelined = n_full % NBUF == 0 and n_full >= 2 * NBUF

    mesh = plsc.VectorSubcoreMesh(core_axis_name="c", subcore_axis_name="s")

    nbuf = NBUF if pipelined else 1
    scratch = (
        [pltpu.VMEM((CHUNK,), jnp.int32) for _ in range(nbuf)] +
        [pltpu.VMEM((CHUNK,), jnp.int32) for _ in range(nbuf)] +
        [pltpu.VMEM((CHUNK,), jnp.float32) for _ in range(nbuf)] +
        [pltpu.VMEM((CHUNK, d), jnp.float32) for _ in range(nbuf)] +
        [pltpu.VMEM((zrows, d), jnp.float32),
         pltpu.VMEM_SHARED((n_nodes, d), jnp.float32)] +
        [pltpu.SemaphoreType.DMA for _ in range(4 * nbuf)]
    )
    if tail:
        scratch += [
            pltpu.VMEM((tail,), jnp.int32),
            pltpu.VMEM((tail,), jnp.int32),
            pltpu.VMEM((tail, d), jnp.float32),
            pltpu.VMEM((tail,), jnp.float32),
        ]

    def body(h_hbm, col_hbm, row_hbm, w_hbm, out_hbm, *refs):
        colv = refs[0:nbuf]
        rowv = refs[nbuf:2 * nbuf]
        wv = refs[2 * nbuf:3 * nbuf]
        rows = refs[3 * nbuf:4 * nbuf]
        zblk = refs[4 * nbuf]
        agg = refs[4 * nbuf + 1]
        sems = refs[4 * nbuf + 2:8 * nbuf + 2]
        scol = sems[0:nbuf]
        srw = sems[nbuf:2 * nbuf]
        sgat = sems[2 * nbuf:3 * nbuf]
        ssc = sems[3 * nbuf:4 * nbuf]
        tail_refs = refs[8 * nbuf + 2:]

        c = lax.axis_index("c")
        s = lax.axis_index("s")

        @pl.loop(0, zrows)
        def _(i):
            for j in range(nv):
                zblk[i, pl.ds(j * LANES, LANES)] = jnp.zeros(
                    (LANES,), jnp.float32)

        base_row = s * rows_per_tile
        for q in range(nz):
            pltpu.sync_copy(zblk, agg.at[pl.ds(base_row + q * zrows, zrows)])
        if leftover:
            @pl.when(s == NS - 1)
            def _():
                pltpu.sync_copy(
                    zblk.at[pl.ds(0, leftover)],
                    agg.at[pl.ds(rows_per_tile * NS, leftover)])
        plsc.subcore_barrier()

        eb = (c * NS + s) * per_tile

        if pipelined:
            for b in range(NBUF):
                pltpu.async_copy(col_hbm.at[pl.ds(eb + b * CHUNK, CHUNK)],
                                 colv[b], scol[b])

            @pl.loop(0, n_full, step=NBUF)
            def _(g):
                for b in range(NBUF):
                    k = g + b

                    @pl.when(k >= NBUF)
                    def _():
                        pltpu.make_async_copy(
                            rows[b], agg.at[rowv[b]], ssc[b]).wait()
                    off = eb + k * CHUNK
                    pltpu.async_copy(row_hbm.at[pl.ds(off, CHUNK)],
                                     rowv[b], srw[b])
                    pltpu.async_copy(w_hbm.at[pl.ds(off, CHUNK)],
                                     wv[b], srw[b])
                    pltpu.make_async_copy(
                        col_hbm.at[pl.ds(off, CHUNK)], colv[b],
                        scol[b]).wait()
                    pltpu.async_copy(h_hbm.at[colv[b]], rows[b], sgat[b])
                for b in range(NBUF):
                    k = g + b
                    off = eb + k * CHUNK
                    pltpu.make_async_copy(
                        h_hbm.at[colv[b]], rows[b], sgat[b]).wait()

                    @pl.when(k + NBUF < n_full)
                    def _():
                        pltpu.async_copy(
                            col_hbm.at[pl.ds(off + NBUF * CHUNK, CHUNK)],
                            colv[b], scol[b])
                    pltpu.make_async_copy(row_hbm.at[pl.ds(off, CHUNK)],
                                          rowv[b], srw[b]).wait()
                    pltpu.make_async_copy(w_hbm.at[pl.ds(off, CHUNK)],
                                          wv[b], srw[b]).wait()
                    _scale_rows(rows[b], wv[b], CHUNK, d)
                    pltpu.async_copy(rows[b], agg.at[rowv[b]], ssc[b],
                                     add=True)

            for b in range(NBUF):
                pltpu.make_async_copy(rows[b], agg.at[rowv[b]], ssc[b]).wait()
        else:
            @pl.loop(0, n_full)
            def _(k):
                off = eb + k * CHUNK
                pltpu.sync_copy(col_hbm.at[pl.ds(off, CHUNK)], colv[0])
                pltpu.sync_copy(w_hbm.at[pl.ds(off, CHUNK)], wv[0])
                pltpu.async_copy(h_hbm.at[colv[0]], rows[0], sgat[0]).wait()
                pltpu.sync_copy(row_hbm.at[pl.ds(off, CHUNK)], rowv[0])
                _scale_rows(rows[0], wv[0], CHUNK, d)
                pltpu.sync_copy(rows[0], agg.at[rowv[0]], add=True)

        if tail:
            tcol, trow, trows, tw = tail_refs
            off = eb + n_full * CHUNK
            pltpu.sync_copy(col_hbm.at[pl.ds(off, tail)], tcol)
            pltpu.sync_copy(w_hbm.at[pl.ds(off, tail)], tw)
            pltpu.async_copy(h_hbm.at[tcol], trows, sgat[0]).wait()
            pltpu.sync_copy(row_hbm.at[pl.ds(off, tail)], trow)
            _scale_rows(trows, tw, tail, d)
            pltpu.sync_copy(trows, agg.at[trow], add=True)

        plsc.subcore_barrier()
        pltpu.sync_copy(agg.at[pl.ds(base_row, rows_per_tile)],
                        out_hbm.at[c, pl.ds(base_row, rows_per_tile)])
        if leftover:
            @pl.when(s == NS - 1)
            def _():
                pltpu.sync_copy(
                    agg.at[pl.ds(rows_per_tile * NS, leftover)],
                    out_hbm.at[c, pl.ds(rows_per_tile * NS, leftover)])

    return pl.kernel(
        body,
        out_type=jax.ShapeDtypeStruct((NC, n_nodes, d), jnp.float32),
        mesh=mesh,
        scratch_types=scratch,
    )


@jax.jit
def kernel(x, edge_index, edge_weight, weight, bias):
    n, d_in = x.shape
    d = weight.shape[1]
    n_edges = edge_weight.shape[0]

    blk = 1000 if n % 1000 == 0 else n
    h = pl.pallas_call(
        _matmul_body,
        grid=(n // blk,),
        in_specs=[
            pl.BlockSpec((blk, d_in), lambda i: (i, 0)),
            pl.BlockSpec((d_in, d), lambda i: (0, 0)),
        ],
        out_specs=pl.BlockSpec((blk, d), lambda i: (i, 0)),
        out_shape=jax.ShapeDtypeStruct((n, d), jnp.float32),
    )(x, weight)

    ei = edge_index.astype(jnp.int32)
    row = ei[0]
    col = ei[1]
    ew = edge_weight.astype(jnp.float32)

    partials = _make_spmm(n, n_edges, d)(h, col, row, ew)

    out = pl.pallas_call(
        _combine_body,
        grid=(n // blk,),
        in_specs=[
            pl.BlockSpec((1, blk, d), lambda i: (0, i, 0)),
            pl.BlockSpec((1, blk, d), lambda i: (1, i, 0)),
            pl.BlockSpec((d,), lambda i: (0,)),
        ],
        out_specs=pl.BlockSpec((blk, d), lambda i: (i, 0)),
        out_shape=jax.ShapeDtypeStruct((n, d), jnp.float32),
    )(partials, partials, bias)
    return out


# speedup vs baseline: 10.8977x; 1.0680x over previous
import functools

import jax
import jax.numpy as jnp
from jax import lax
from jax.experimental import pallas as pl
from jax.experimental.pallas import tpu as pltpu
from jax.experimental.pallas import tpu_sc as plsc

NC = 2
NS = 16
LANES = 16
CHUNK = 64


def _matmul_body(x_ref, w_ref, o_ref):
    o_ref[...] = jnp.dot(x_ref[...], w_ref[...],
                         preferred_element_type=jnp.float32)


def _combine_body(a_ref, b_ref, bias_ref, o_ref):
    o_ref[...] = a_ref[0] + b_ref[0] + bias_ref[...]


def _scale_rows(rows_ref, w_ref, n_edges, d):
    nv = d // LANES
    assert n_edges % LANES == 0

    @pl.loop(0, n_edges // LANES)
    def _(g):
        wv16 = w_ref[pl.ds(g * LANES, LANES)]
        for l in range(LANES):
            wb = jnp.full((LANES,), wv16[l], dtype=jnp.float32)
            e = g * LANES + l
            for j in range(nv):
                sl = pl.ds(j * LANES, LANES)
                rows_ref[e, sl] = rows_ref[e, sl] * wb


def _make_spmm(n_nodes, n_edges, d):
    per_tile = n_edges // (NC * NS)
    n_full = per_tile // CHUNK
    tail = per_tile - n_full * CHUNK
    rows_per_tile = (n_nodes // NS) // 8 * 8
    leftover = n_nodes - rows_per_tile * NS
    assert leftover % 8 == 0
    zrows = rows_per_tile
    for cand in (64, 56, 48, 40, 32, 24, 16, 8):
        if rows_per_tile % cand == 0:
            zrows = cand
            break
    nz = rows_per_tile // zrows
    assert leftover <= zrows
    nv = d // LANES

    NBUF = 4
    pipelined = n_full % NBUF == 0 and n_full >= 2 * NBUF

    mesh = plsc.VectorSubcoreMesh(core_axis_name="c", subcore_axis_name="s")

    nbuf = NBUF if pipelined else 1
    scratch = (
        [pltpu.VMEM((CHUNK,), jnp.int32) for _ in range(nbuf)] +
        [pltpu.VMEM((CHUNK,), jnp.int32) for _ in range(nbuf)] +
        [pltpu.VMEM((CHUNK,), jnp.float32) for _ in range(nbuf)] +
        [pltpu.VMEM((CHUNK, d), jnp.float32) for _ in range(nbuf)] +
        [pltpu.VMEM((zrows, d), jnp.float32),
         pltpu.VMEM_SHARED((n_nodes, d), jnp.float32)] +
        [pltpu.SemaphoreType.DMA for _ in range(4 * nbuf)]
    )
    if tail:
        scratch += [
            pltpu.VMEM((tail,), jnp.int32),
            pltpu.VMEM((tail,), jnp.int32),
            pltpu.VMEM((tail, d), jnp.float32),
            pltpu.VMEM((tail,), jnp.float32),
        ]

    def body(h_hbm, col_hbm, row_hbm, w_hbm, out_hbm, *refs):
        colv = refs[0:nbuf]
        rowv = refs[nbuf:2 * nbuf]
        wv = refs[2 * nbuf:3 * nbuf]
        rows = refs[3 * nbuf:4 * nbuf]
        zblk = refs[4 * nbuf]
        agg = refs[4 * nbuf + 1]
        sems = refs[4 * nbuf + 2:8 * nbuf + 2]
        scol = sems[0:nbuf]
        srw = sems[nbuf:2 * nbuf]
        sgat = sems[2 * nbuf:3 * nbuf]
        ssc = sems[3 * nbuf:4 * nbuf]
        tail_refs = refs[8 * nbuf + 2:]

        c = lax.axis_index("c")
        s = lax.axis_index("s")

        @pl.loop(0, zrows)
        def _(i):
            for j in range(nv):
                zblk[i, pl.ds(j * LANES, LANES)] = jnp.zeros(
                    (LANES,), jnp.float32)

        base_row = s * rows_per_tile
        for q in range(nz):
            pltpu.sync_copy(zblk, agg.at[pl.ds(base_row + q * zrows, zrows)])
        if leftover:
            @pl.when(s == NS - 1)
            def _():
                pltpu.sync_copy(
                    zblk.at[pl.ds(0, leftover)],
                    agg.at[pl.ds(rows_per_tile * NS, leftover)])
        plsc.subcore_barrier()

        eb = (c * NS + s) * per_tile

        if pipelined:
            for b in range(NBUF):
                pltpu.async_copy(col_hbm.at[pl.ds(eb + b * CHUNK, CHUNK)],
                                 colv[b], scol[b])

            @pl.loop(0, n_full, step=NBUF)
            def _(g):
                for b in range(NBUF):
                    k = g + b

                    @pl.when(k >= NBUF)
                    def _():
                        pltpu.make_async_copy(
                            rows[b], agg.at[rowv[b]], ssc[b]).wait()
                    off = eb + k * CHUNK
                    pltpu.async_copy(row_hbm.at[pl.ds(off, CHUNK)],
                                     rowv[b], srw[b])
                    pltpu.async_copy(w_hbm.at[pl.ds(off, CHUNK)],
                                     wv[b], srw[b])
                    pltpu.make_async_copy(
                        col_hbm.at[pl.ds(off, CHUNK)], colv[b],
                        scol[b]).wait()
                    pltpu.async_copy(h_hbm.at[colv[b]], rows[b], sgat[b])
                for b in range(NBUF):
                    k = g + b
                    off = eb + k * CHUNK
                    pltpu.make_async_copy(
                        h_hbm.at[colv[b]], rows[b], sgat[b]).wait()

                    @pl.when(k + NBUF < n_full)
                    def _():
                        pltpu.async_copy(
                            col_hbm.at[pl.ds(off + NBUF * CHUNK, CHUNK)],
                            colv[b], scol[b])
                    pltpu.make_async_copy(row_hbm.at[pl.ds(off, CHUNK)],
                                          rowv[b], srw[b]).wait()
                    pltpu.make_async_copy(w_hbm.at[pl.ds(off, CHUNK)],
                                          wv[b], srw[b]).wait()
                    _scale_rows(rows[b], wv[b], CHUNK, d)
                    pltpu.async_copy(rows[b], agg.at[rowv[b]], ssc[b],
                                     add=True)

            for b in range(NBUF):
                pltpu.make_async_copy(rows[b], agg.at[rowv[b]], ssc[b]).wait()
        else:
            @pl.loop(0, n_full)
            def _(k):
                off = eb + k * CHUNK
                pltpu.sync_copy(col_hbm.at[pl.ds(off, CHUNK)], colv[0])
                pltpu.sync_copy(w_hbm.at[pl.ds(off, CHUNK)], wv[0])
                pltpu.async_copy(h_hbm.at[colv[0]], rows[0], sgat[0]).wait()
                pltpu.sync_copy(row_hbm.at[pl.ds(off, CHUNK)], rowv[0])
                _scale_rows(rows[0], wv[0], CHUNK, d)
                pltpu.sync_copy(rows[0], agg.at[rowv[0]], add=True)

        if tail:
            tcol, trow, trows, tw = tail_refs
            off = eb + n_full * CHUNK
            pltpu.sync_copy(col_hbm.at[pl.ds(off, tail)], tcol)
            pltpu.sync_copy(w_hbm.at[pl.ds(off, tail)], tw)
            pltpu.async_copy(h_hbm.at[tcol], trows, sgat[0]).wait()
            pltpu.sync_copy(row_hbm.at[pl.ds(off, tail)], trow)
            _scale_rows(trows, tw, tail, d)
            pltpu.sync_copy(trows, agg.at[trow], add=True)

        plsc.subcore_barrier()
        pltpu.sync_copy(agg.at[pl.ds(base_row, rows_per_tile)],
                        out_hbm.at[c, pl.ds(base_row, rows_per_tile)])
        if leftover:
            @pl.when(s == NS - 1)
            def _():
                pltpu.sync_copy(
                    agg.at[pl.ds(rows_per_tile * NS, leftover)],
                    out_hbm.at[c, pl.ds(rows_per_tile * NS, leftover)])

    return pl.kernel(
        body,
        out_type=jax.ShapeDtypeStruct((NC, n_nodes, d), jnp.float32),
        mesh=mesh,
        scratch_types=scratch,
    )


@jax.jit
def kernel(x, edge_index, edge_weight, weight, bias):
    n, d_in = x.shape
    d = weight.shape[1]
    n_edges = edge_weight.shape[0]

    blk = 1000 if n % 1000 == 0 else n
    h = pl.pallas_call(
        _matmul_body,
        grid=(n // blk,),
        in_specs=[
            pl.BlockSpec((blk, d_in), lambda i: (i, 0)),
            pl.BlockSpec((d_in, d), lambda i: (0, 0)),
        ],
        out_specs=pl.BlockSpec((blk, d), lambda i: (i, 0)),
        out_shape=jax.ShapeDtypeStruct((n, d), jnp.float32),
    )(x, weight)

    ei = edge_index.astype(jnp.int32)
    row = ei[0]
    col = ei[1]
    ew = edge_weight.astype(jnp.float32)

    partials = _make_spmm(n, n_edges, d)(h, col, row, ew)

    out = pl.pallas_call(
        _combine_body,
        grid=(n // blk,),
        in_specs=[
            pl.BlockSpec((1, blk, d), lambda i: (0, i, 0)),
            pl.BlockSpec((1, blk, d), lambda i: (1, i, 0)),
            pl.BlockSpec((d,), lambda i: (0,)),
        ],
        out_specs=pl.BlockSpec((blk, d), lambda i: (i, 0)),
        out_shape=jax.ShapeDtypeStruct((n, d), jnp.float32),
    )(partials, partials, bias)
    return out
